# baseline (speedup 1.0000x reference)
import functools

import jax
import jax.numpy as jnp
from jax import lax
from jax.experimental import pallas as pl
from jax.experimental.pallas import tpu as pltpu
from jax.experimental.pallas import tpu_sc as plsc

_N = 100000
_ZMAX = 87
_F = 128

_NC = 2
_NS = 16
_NW = _NC * _NS
_C = 320
_K = 10
_W = _K * _C
_LAST_BASE = _N - _W


def _table_body(emb_ref, ec_ref, cw_ref, out_ref):
    out_ref[...] = emb_ref[...] + lax.dot_general(
        ec_ref[...], cw_ref[...], (((1,), (1,)), ((), ())),
        preferred_element_type=jnp.float32)


_table_call = pl.pallas_call(
    _table_body,
    out_shape=jax.ShapeDtypeStruct((_ZMAX, _F), jnp.float32),
)


_sc_mesh = plsc.VectorSubcoreMesh(core_axis_name="c", subcore_axis_name="s")


_NBUF = 2


@functools.partial(
    pl.kernel,
    mesh=_sc_mesh,
    out_type=jax.ShapeDtypeStruct((_N * _F,), jnp.float32),
    scratch_types=[
        pltpu.VMEM((_ZMAX * _F,), jnp.float32),
        pltpu.VMEM((_W,), jnp.int32),
    ]
    + [pltpu.VMEM((_C * _F,), jnp.float32) for _ in range(_NBUF)]
    + [pltpu.SemaphoreType.DMA for _ in range(_NBUF)]
    + [pltpu.SemaphoreType.DMA],
    compiler_params=pltpu.CompilerParams(needs_layout_passes=False),
)
def _gather_kernel(table_hbm, z_hbm, out_hbm, table_v, idx_v, *bufs_and_sems):
    stag = bufs_and_sems[:_NBUF]
    wsem = bufs_and_sems[_NBUF:2 * _NBUF]
    lsem = bufs_and_sems[2 * _NBUF]
    wid = lax.axis_index("s") * _NC + lax.axis_index("c")
    row_base = lax.min(wid * _W, _LAST_BASE)
    pltpu.async_copy(table_hbm, table_v, lsem)
    pltpu.async_copy(z_hbm.at[pl.ds(row_base, _W)], idx_v, lsem)
    pltpu.make_async_copy(table_hbm, table_v, lsem).wait()
    pltpu.make_async_copy(z_hbm.at[pl.ds(row_base, _W)], idx_v, lsem).wait()
    base = row_base * _F
    iota16 = lax.broadcasted_iota(jnp.int32, (16,), 0)
    ktab = [iota16 + 16 * k for k in range(_F // 16)]

    def compute_chunk(c, b):
        @plsc.parallel_loop(0, _C // 16)
        def body_p(p):
            zv = idx_v[pl.ds(c * _C + p * 16, 16)]
            zb = zv * _F
            for i in range(16):
                zi = jnp.take_along_axis(
                    zb, jnp.full((16,), i, jnp.int32), axis=0,
                    mode="promise_in_bounds")
                row_off = p * (16 * _F) + i * _F
                for k in range(_F // 16):
                    vals = plsc.load_gather(table_v, [zi + ktab[k]])
                    stag[b][pl.ds(row_off + 16 * k, 16)] = vals

    def out_slice(c):
        return out_hbm.at[pl.ds(base + c * (_C * _F), _C * _F)]

    def fire_write(c, b):
        pltpu.async_copy(stag[b], out_slice(c), wsem[b])

    def wait_write(c, b):
        pltpu.make_async_copy(stag[b], out_slice(c), wsem[b]).wait()

    def body(j, carry):
        for b in range(_NBUF):
            c = j * _NBUF + b

            @pl.when(j > 0)
            def _():
                wait_write(c - _NBUF, b)

            compute_chunk(c, b)
            fire_write(c, b)
        return carry

    lax.fori_loop(0, _K // _NBUF, body, 0)

    for b in range(_NBUF):
        wait_write(_K - _NBUF + b, b)


def kernel(Z, element_embedding, electron_config, config_weight):
    table = _table_call(element_embedding, electron_config, config_weight)
    return _gather_kernel(table.reshape(-1), Z.astype(jnp.int32)).reshape(_N, _F)


# speedup vs baseline: 1.0892x; 1.0892x over previous
import functools

import jax
import jax.numpy as jnp
from jax import lax
from jax.experimental import pallas as pl
from jax.experimental.pallas import tpu as pltpu
from jax.experimental.pallas import tpu_sc as plsc

_N = 100000
_ZMAX = 87
_F = 128

_NC = 2
_NS = 16
_NW = _NC * _NS
_C = 400
_K = 8
_W = _K * _C
_LAST_BASE = _N - _W


def _table_body(emb_ref, ec_ref, cw_ref, out_ref):
    out_ref[...] = emb_ref[...] + lax.dot_general(
        ec_ref[...], cw_ref[...], (((1,), (1,)), ((), ())),
        preferred_element_type=jnp.float32)


_table_call = pl.pallas_call(
    _table_body,
    out_shape=jax.ShapeDtypeStruct((_ZMAX, _F), jnp.float32),
)


_sc_mesh = plsc.VectorSubcoreMesh(core_axis_name="c", subcore_axis_name="s")


_NBUF = 2


@functools.partial(
    pl.kernel,
    mesh=_sc_mesh,
    out_type=jax.ShapeDtypeStruct((_N * _F,), jnp.float32),
    scratch_types=[
        pltpu.VMEM((_ZMAX * _F,), jnp.float32),
        pltpu.VMEM((_W,), jnp.int32),
    ]
    + [pltpu.VMEM((_C * _F,), jnp.float32) for _ in range(_NBUF)]
    + [pltpu.SemaphoreType.DMA for _ in range(_NBUF)]
    + [pltpu.SemaphoreType.DMA],
    compiler_params=pltpu.CompilerParams(needs_layout_passes=False),
)
def _gather_kernel(table_hbm, z_hbm, out_hbm, table_v, idx_v, *bufs_and_sems):
    stag = bufs_and_sems[:_NBUF]
    wsem = bufs_and_sems[_NBUF:2 * _NBUF]
    lsem = bufs_and_sems[2 * _NBUF]
    wid = lax.axis_index("s") * _NC + lax.axis_index("c")
    row_base = lax.min(wid * _W, _LAST_BASE)
    pltpu.async_copy(table_hbm, table_v, lsem)
    pltpu.async_copy(z_hbm.at[pl.ds(row_base, _W)], idx_v, lsem)
    pltpu.make_async_copy(table_hbm, table_v, lsem).wait()
    pltpu.make_async_copy(z_hbm.at[pl.ds(row_base, _W)], idx_v, lsem).wait()
    base = row_base * _F
    iota16 = lax.broadcasted_iota(jnp.int32, (16,), 0)
    ktab = [iota16 + 16 * k for k in range(_F // 16)]

    def compute_chunk(c, b):
        @plsc.parallel_loop(0, _C // 16)
        def body_p(p):
            zv = idx_v[pl.ds(c * _C + p * 16, 16)]
            zb = zv * _F
            for i in range(16):
                zi = jnp.take_along_axis(
                    zb, jnp.full((16,), i, jnp.int32), axis=0,
                    mode="promise_in_bounds")
                row_off = p * (16 * _F) + i * _F
                for k in range(_F // 16):
                    vals = plsc.load_gather(table_v, [zi + ktab[k]])
                    stag[b][pl.ds(row_off + 16 * k, 16)] = vals

    def out_slice(c):
        return out_hbm.at[pl.ds(base + c * (_C * _F), _C * _F)]

    def fire_write(c, b):
        pltpu.async_copy(stag[b], out_slice(c), wsem[b])

    def wait_write(c, b):
        pltpu.make_async_copy(stag[b], out_slice(c), wsem[b]).wait()

    def body(j, carry):
        for b in range(_NBUF):
            c = j * _NBUF + b

            @pl.when(j > 0)
            def _():
                wait_write(c - _NBUF, b)

            compute_chunk(c, b)
            fire_write(c, b)
        return carry

    lax.fori_loop(0, _K // _NBUF, body, 0)

    for b in range(_NBUF):
        wait_write(_K - _NBUF + b, b)


def kernel(Z, element_embedding, electron_config, config_weight):
    table = _table_call(element_embedding, electron_config, config_weight)
    return _gather_kernel(table.reshape(-1), Z.astype(jnp.int32)).reshape(_N, _F)
